# baseline (speedup 1.0000x reference)
import jax
import jax.numpy as jnp
from jax import lax
from jax.experimental import pallas as pl
from jax.experimental.pallas import tpu as pltpu
from jax.experimental.pallas import tpu_sc as plsc

N_USERS = 20000
N_ITEMS = 30000
N_NODES = N_USERS + N_ITEMS
K = 64
KH = 32

NC = 2
NS = 16

U = 128
UNITS = 6400
U_PER_TILE = UNITS // NS
U_CH = 50
N_CH = U_PER_TILE // U_CH
NB = 4
GRP = (2 * U_CH) // NB

ACC_ROWS = 50176
ZCH = 112
RO_CH = 125
TBL_ROWS = 50176

HALF = N_NODES // NC
DEG_ROWS = 25088
DEG_RO = 1568
DUMMY0 = 25024
DEG_W = 16


def _sc_mesh():
    return plsc.VectorSubcoreMesh(core_axis_name="c", subcore_axis_name="s")


_SC_PARAMS = pltpu.CompilerParams(use_tc_tiling_on_sc=False)


def _deg_body(ep_hbm, out_hbm, dst_v, ones_v, stg, acc, sem):
    h = lax.axis_index("c")
    s = lax.axis_index("s")
    base = h * HALF

    one = jnp.full((16,), 1.0, jnp.float32)

    def fill(r, _):
        ones_v[r, pl.ds(0, 16)] = one
        return 0

    lax.fori_loop(0, U, fill, 0)

    zero = jnp.zeros((16,), jnp.float32)

    def zr(r, _):
        stg[r, pl.ds(0, 16)] = zero
        return 0

    lax.fori_loop(0, DEG_RO, zr, 0)
    pltpu.sync_copy(stg, acc.at[pl.ds(s * DEG_RO, DEG_RO)])
    plsc.subcore_barrier()

    def outer(oc, _):
        u0 = s * U_PER_TILE + oc * U_CH
        for d in range(2):
            pltpu.sync_copy(ep_hbm.at[1 - d, pl.ds(u0, U_CH)], dst_v)

            def inner(k, _):
                for c in range(U // 16):
                    dd = dst_v[k, pl.ds(c * 16, 16)]
                    inr = (dd >= base) & (dd < base + HALF)
                    dum = DUMMY0 + (c % 4) * 16 + lax.iota(jnp.int32, 16)
                    dst_v[k, pl.ds(c * 16, 16)] = jnp.where(inr, dd - base, dum)
                pltpu.sync_copy(ones_v, acc.at[dst_v.at[k]], add=True)
                return 0

            lax.fori_loop(0, U_CH, inner, 0)
        return 0

    lax.fori_loop(0, N_CH, outer, 0)
    plsc.subcore_barrier()

    pltpu.sync_copy(acc.at[pl.ds(s * DEG_RO, DEG_RO)], stg)
    pltpu.sync_copy(stg, out_hbm.at[h, pl.ds(s * DEG_RO, DEG_RO)])


_deg_call = pl.kernel(
    _deg_body,
    out_type=jax.ShapeDtypeStruct((NC, DEG_ROWS, DEG_W), jnp.float32),
    mesh=_sc_mesh(),
    scratch_types=[
        pltpu.VMEM((U_CH, U), jnp.int32),
        pltpu.VMEM((U, DEG_W), jnp.float32),
        pltpu.VMEM((DEG_RO, DEG_W), jnp.float32),
        pltpu.VMEM_SHARED((DEG_ROWS, DEG_W), jnp.float32),
        pltpu.SemaphoreType.DMA,
    ],
    compiler_params=_SC_PARAMS,
)


def _mega_body(ep_hbm, xh0_hbm, dinv_hbm, xh1_hbm, xh2_hbm, z3_hbm,
               e0_v, e1_v, r0_v, r1_v, r2_v, r3_v, acc_ref,
               sem0, sem1, sem2, sem3, ssem0, ssem1, ssem2, ssem3):
    h = lax.axis_index("c")
    s = lax.axis_index("s")
    rows = (r0_v, r1_v, r2_v, r3_v)
    sems = (sem0, sem1, sem2, sem3)
    ssems = (ssem0, ssem1, ssem2, ssem3)

    def zero_r3():
        zero = jnp.zeros((16,), jnp.float32)

        def zr(r, _):
            for c in range(KH // 16):
                r3_v[r, pl.ds(c * 16, 16)] = zero
            return 0

        lax.fori_loop(0, U, zr, 0)

    def body_with_acc(acc):
        zero_r3()

        def zcp(i, _):
            pltpu.sync_copy(r3_v.at[pl.ds(0, ZCH)],
                            acc.at[pl.ds(s * (ACC_ROWS // NS) + i * ZCH, ZCH)])
            return 0

        lax.fori_loop(0, ACC_ROWS // (NS * ZCH), zcp, 0)
        plsc.subcore_barrier()

        def scatter_phase(table_hbm):
            def chunk(oc, _):
                u0 = s * U_PER_TILE + oc * U_CH
                pltpu.sync_copy(ep_hbm.at[0, pl.ds(u0, U_CH)], e0_v)
                pltpu.sync_copy(ep_hbm.at[1, pl.ds(u0, U_CH)], e1_v)

                def fire(g, j):
                    k = 2 * g + (j // 2)
                    gsrc = e0_v if j % 2 == 0 else e1_v
                    pltpu.async_copy(table_hbm.at[gsrc.at[k]], rows[j],
                                     sems[j])

                def scat(g, j):
                    k = 2 * g + (j // 2)
                    gdst = e1_v if j % 2 == 0 else e0_v
                    pltpu.async_copy(rows[j], acc.at[gdst.at[k]], ssems[j],
                                     add=True)

                def wait_scat(j):
                    pltpu.make_async_copy(
                        rows[j], acc.at[pl.ds(0, U)], ssems[j]
                    ).wait()

                for j in range(NB):
                    fire(0, j)

                def grp(g, _):
                    for j in range(NB):
                        pltpu.make_async_copy(
                            table_hbm.at[pl.ds(0, U)], rows[j], sems[j]
                        ).wait()
                        scat(g, j)

                    for j in range(NB):
                        @pl.when(g < GRP - 1)
                        def _():
                            wait_scat(j)
                            fire(g + 1, j)

                    return 0

                lax.fori_loop(0, GRP, grp, 0)
                for j in range(NB):
                    wait_scat(j)
                return 0

            lax.fori_loop(0, N_CH, chunk, 0)
            plsc.subcore_barrier()

        def readout_phase(out_hbm, scale, rezero):
            if rezero:
                zero_r3()

            def ro(i, _):
                r0 = s * (N_NODES // NS) + i * RO_CH
                pltpu.sync_copy(acc.at[pl.ds(r0, RO_CH)],
                                r0_v.at[pl.ds(0, RO_CH)])
                if scale:
                    pltpu.sync_copy(dinv_hbm.at[pl.ds(r0, RO_CH)],
                                    r1_v.at[pl.ds(0, RO_CH)])

                    def mrow(r, _):
                        for c in range(KH // 16):
                            sl = pl.ds(c * 16, 16)
                            r0_v[r, sl] = r0_v[r, sl] * r1_v[r, sl]
                        return 0

                    lax.fori_loop(0, RO_CH, mrow, 0)
                pltpu.sync_copy(r0_v.at[pl.ds(0, RO_CH)],
                                out_hbm.at[h, pl.ds(r0, RO_CH)])
                if rezero:
                    pltpu.sync_copy(r3_v.at[pl.ds(0, RO_CH)],
                                    acc.at[pl.ds(r0, RO_CH)])
                return 0

            lax.fori_loop(0, N_NODES // (NS * RO_CH), ro, 0)
            plsc.subcore_barrier()

        scatter_phase(xh0_hbm.at[h])
        readout_phase(xh1_hbm, scale=True, rezero=True)
        scatter_phase(xh1_hbm.at[h])
        readout_phase(xh2_hbm, scale=True, rezero=True)
        scatter_phase(xh2_hbm.at[h])
        readout_phase(z3_hbm, scale=False, rezero=False)

    body_with_acc(acc_ref)


_mega_call = pl.kernel(
    _mega_body,
    out_type=(
        jax.ShapeDtypeStruct((NC, TBL_ROWS, KH), jnp.float32),
        jax.ShapeDtypeStruct((NC, TBL_ROWS, KH), jnp.float32),
        jax.ShapeDtypeStruct((NC, TBL_ROWS, KH), jnp.float32),
    ),
    mesh=_sc_mesh(),
    scratch_types=[
        pltpu.VMEM((U_CH, U), jnp.int32),
        pltpu.VMEM((U_CH, U), jnp.int32),
        pltpu.VMEM((U, KH), jnp.float32),
        pltpu.VMEM((U, KH), jnp.float32),
        pltpu.VMEM((U, KH), jnp.float32),
        pltpu.VMEM((U, KH), jnp.float32),
        pltpu.VMEM_SHARED((ACC_ROWS, KH), jnp.float32),
        pltpu.SemaphoreType.DMA,
        pltpu.SemaphoreType.DMA,
        pltpu.SemaphoreType.DMA,
        pltpu.SemaphoreType.DMA,
        pltpu.SemaphoreType.DMA,
        pltpu.SemaphoreType.DMA,
        pltpu.SemaphoreType.DMA,
        pltpu.SemaphoreType.DMA,
    ],
    compiler_params=_SC_PARAMS,
)


TC_B = 200
TC_GRID = (NC, HALF // TC_B)


def _deg_spec():
    return pl.BlockSpec((1, TC_B, DEG_W), lambda h, i: (h, i, 0))


def _ego_spec():
    return pl.BlockSpec((1, TC_B, K), lambda h, i: (h, i, 0))


def _col_spec():
    return pl.BlockSpec((NC, TC_B, KH), lambda h, i: (0, h * (HALF // TC_B) + i, 0))


def _flat_spec():
    return pl.BlockSpec((TC_B, KH), lambda h, i: (h * (HALF // TC_B) + i, 0))


def _prep_body(deg_ref, ego_ref, xh0_ref, dinv_ref):
    d = jnp.maximum(deg_ref[0, :, 0:1], 1.0)
    rs = lax.rsqrt(d)
    xh0_ref[0] = ego_ref[0, :, :KH] * rs
    xh0_ref[1] = ego_ref[0, :, KH:] * rs
    dinv_ref[...] = jnp.broadcast_to(1.0 / d, (TC_B, KH))


def _final_body(deg_ref, ego_ref, xh1_ref, xh2_ref, z3_ref, out_ref):
    d = jnp.maximum(deg_ref[0, :, 0:1], 1.0)
    rs = lax.rsqrt(d)
    for p in range(NC):
        zs = (xh1_ref[p] + xh2_ref[p]) * d + z3_ref[p]
        out_ref[0, :, p * KH:(p + 1) * KH] = (
            ego_ref[0, :, p * KH:(p + 1) * KH] + zs * rs) * 0.25


_prep_call = pl.pallas_call(
    _prep_body,
    grid=TC_GRID,
    in_specs=[_deg_spec(), _ego_spec()],
    out_specs=(_col_spec(), _flat_spec()),
    out_shape=(
        jax.ShapeDtypeStruct((NC, TBL_ROWS, KH), jnp.float32),
        jax.ShapeDtypeStruct((N_NODES, KH), jnp.float32),
    ),
)

_final_call = pl.pallas_call(
    _final_body,
    grid=TC_GRID,
    in_specs=[_deg_spec(), _ego_spec(), _col_spec(), _col_spec(), _col_spec()],
    out_specs=_ego_spec(),
    out_shape=jax.ShapeDtypeStruct((NC, HALF, K), jnp.float32),
)


def kernel(edge_index, Gu, Gi):
    e0 = edge_index[0].astype(jnp.int32)
    e1 = edge_index[1].astype(jnp.int32)
    npad = UNITS * U - e0.shape[0]
    ar = jnp.arange(npad, dtype=jnp.int32)
    pe0 = jnp.concatenate([e0, N_NODES + (ar % 64)])
    pe1 = jnp.concatenate([e1, N_NODES + 64 + (ar % 64)])
    ep = jnp.stack([pe0, pe1]).reshape(2, UNITS, U)
    ego = jnp.concatenate([Gu, Gi], axis=0)
    ego3d = ego.reshape(NC, HALF, K)

    deg = _deg_call(ep)
    xh0, dinv = _prep_call(deg, ego3d)
    xh1, xh2, z3 = _mega_call(ep, xh0, dinv)
    out = _final_call(deg, ego3d, xh1, xh2, z3)
    out = out.reshape(N_NODES, K)
    return out[:N_USERS], out[N_USERS:]


# speedup vs baseline: 1.0774x; 1.0774x over previous
import jax
import jax.numpy as jnp
from jax import lax
from jax.experimental import pallas as pl
from jax.experimental.pallas import tpu as pltpu
from jax.experimental.pallas import tpu_sc as plsc

N_USERS = 20000
N_ITEMS = 30000
N_NODES = N_USERS + N_ITEMS
K = 64
KH = 32

NC = 2
NS = 16

U = 128
UNITS = 6400
U_PER_TILE = UNITS // NS
U_CH = 50
N_CH = U_PER_TILE // U_CH
NB = 4
GRP = (2 * U_CH) // NB

ACC_ROWS = 50176
ZCH = 112
RO_CH = 125
TBL_ROWS = 50176

HALF = N_NODES // NC
DEG_ROWS = 25088
DEG_RO = 1568
DUMMY0 = 25024
DEG_W = 16


def _sc_mesh():
    return plsc.VectorSubcoreMesh(core_axis_name="c", subcore_axis_name="s")


_SC_PARAMS = pltpu.CompilerParams(use_tc_tiling_on_sc=False)


def _deg_body(ep_hbm, out_hbm, dst_v, ones_v, stg, acc, sem):
    h = lax.axis_index("c")
    s = lax.axis_index("s")
    base = h * HALF

    one = jnp.full((16,), 1.0, jnp.float32)

    def fill(r, _):
        ones_v[r, pl.ds(0, 16)] = one
        return 0

    lax.fori_loop(0, U, fill, 0)

    zero = jnp.zeros((16,), jnp.float32)

    def zr(r, _):
        stg[r, pl.ds(0, 16)] = zero
        return 0

    lax.fori_loop(0, DEG_RO, zr, 0)
    pltpu.sync_copy(stg, acc.at[pl.ds(s * DEG_RO, DEG_RO)])
    plsc.subcore_barrier()

    def outer(oc, _):
        u0 = s * U_PER_TILE + oc * U_CH
        for d in range(2):
            pltpu.sync_copy(ep_hbm.at[1 - d, pl.ds(u0, U_CH)], dst_v)

            def inner(k, _):
                for c in range(U // 16):
                    dd = dst_v[k, pl.ds(c * 16, 16)]
                    inr = (dd >= base) & (dd < base + HALF)
                    dum = DUMMY0 + (c % 4) * 16 + lax.iota(jnp.int32, 16)
                    dst_v[k, pl.ds(c * 16, 16)] = jnp.where(inr, dd - base, dum)
                pltpu.sync_copy(ones_v, acc.at[dst_v.at[k]], add=True)
                return 0

            lax.fori_loop(0, U_CH, inner, 0)
        return 0

    lax.fori_loop(0, N_CH, outer, 0)
    plsc.subcore_barrier()

    pltpu.sync_copy(acc.at[pl.ds(s * DEG_RO, DEG_RO)], stg)
    pltpu.sync_copy(stg, out_hbm.at[h, pl.ds(s * DEG_RO, DEG_RO)])


_deg_call = pl.kernel(
    _deg_body,
    out_type=jax.ShapeDtypeStruct((NC, DEG_ROWS, DEG_W), jnp.float32),
    mesh=_sc_mesh(),
    scratch_types=[
        pltpu.VMEM((U_CH, U), jnp.int32),
        pltpu.VMEM((U, DEG_W), jnp.float32),
        pltpu.VMEM((DEG_RO, DEG_W), jnp.float32),
        pltpu.VMEM_SHARED((DEG_ROWS, DEG_W), jnp.float32),
        pltpu.SemaphoreType.DMA,
    ],
    compiler_params=_SC_PARAMS,
)


def _mega_body(ep_hbm, xh0_hbm, dinv_hbm, xh1_hbm, xh2_hbm, z3_hbm,
               e0_v, e1_v, r0_v, r1_v, r2_v, r3_v, acc_ref,
               sem0, sem1, sem2, sem3, ssem0, ssem1, ssem2, ssem3):
    h = lax.axis_index("c")
    s = lax.axis_index("s")
    rows = (r0_v, r1_v, r2_v, r3_v)
    sems = (sem0, sem1, sem2, sem3)
    ssems = (ssem0, ssem1, ssem2, ssem3)

    def zero_r3():
        zero = jnp.zeros((16,), jnp.float32)

        def zr(r, _):
            for c in range(KH // 16):
                r3_v[r, pl.ds(c * 16, 16)] = zero
            return 0

        lax.fori_loop(0, U, zr, 0)

    def body_with_acc(acc):
        zero_r3()

        def zcp(i, _):
            pltpu.sync_copy(r3_v.at[pl.ds(0, ZCH)],
                            acc.at[pl.ds(s * (ACC_ROWS // NS) + i * ZCH, ZCH)])
            return 0

        lax.fori_loop(0, ACC_ROWS // (NS * ZCH), zcp, 0)
        plsc.subcore_barrier()

        def scatter_phase(table_hbm):
            def chunk(oc, _):
                u0 = s * U_PER_TILE + oc * U_CH
                pltpu.sync_copy(ep_hbm.at[0, pl.ds(u0, U_CH)], e0_v)
                pltpu.sync_copy(ep_hbm.at[1, pl.ds(u0, U_CH)], e1_v)

                def fire(g, j):
                    k = 2 * g + (j // 2)
                    gsrc = e0_v if j % 2 == 0 else e1_v
                    pltpu.async_copy(table_hbm.at[gsrc.at[k]], rows[j],
                                     sems[j])

                def scat(g, j):
                    k = 2 * g + (j // 2)
                    gdst = e1_v if j % 2 == 0 else e0_v
                    pltpu.sync_copy(rows[j], acc.at[gdst.at[k]], add=True)

                for j in range(NB):
                    fire(0, j)

                def grp(g, _):
                    for j in range(NB):
                        pltpu.make_async_copy(
                            table_hbm.at[pl.ds(0, U)], rows[j], sems[j]
                        ).wait()
                        scat(g, j)

                        @pl.when(g < GRP - 1)
                        def _():
                            fire(g + 1, j)

                    return 0

                lax.fori_loop(0, GRP, grp, 0)
                return 0

            lax.fori_loop(0, N_CH, chunk, 0)
            plsc.subcore_barrier()

        def readout_phase(out_hbm, scale, rezero):
            if rezero:
                zero_r3()

            def ro(i, _):
                r0 = s * (N_NODES // NS) + i * RO_CH
                pltpu.sync_copy(acc.at[pl.ds(r0, RO_CH)],
                                r0_v.at[pl.ds(0, RO_CH)])
                if scale:
                    pltpu.sync_copy(dinv_hbm.at[pl.ds(r0, RO_CH)],
                                    r1_v.at[pl.ds(0, RO_CH)])

                    def mrow(r, _):
                        for c in range(KH // 16):
                            sl = pl.ds(c * 16, 16)
                            r0_v[r, sl] = r0_v[r, sl] * r1_v[r, sl]
                        return 0

                    lax.fori_loop(0, RO_CH, mrow, 0)
                pltpu.sync_copy(r0_v.at[pl.ds(0, RO_CH)],
                                out_hbm.at[h, pl.ds(r0, RO_CH)])
                if rezero:
                    pltpu.sync_copy(r3_v.at[pl.ds(0, RO_CH)],
                                    acc.at[pl.ds(r0, RO_CH)])
                return 0

            lax.fori_loop(0, N_NODES // (NS * RO_CH), ro, 0)
            plsc.subcore_barrier()

        scatter_phase(xh0_hbm.at[h])
        readout_phase(xh1_hbm, scale=True, rezero=True)
        scatter_phase(xh1_hbm.at[h])
        readout_phase(xh2_hbm, scale=True, rezero=True)
        scatter_phase(xh2_hbm.at[h])
        readout_phase(z3_hbm, scale=False, rezero=False)

    body_with_acc(acc_ref)


_mega_call = pl.kernel(
    _mega_body,
    out_type=(
        jax.ShapeDtypeStruct((NC, TBL_ROWS, KH), jnp.float32),
        jax.ShapeDtypeStruct((NC, TBL_ROWS, KH), jnp.float32),
        jax.ShapeDtypeStruct((NC, TBL_ROWS, KH), jnp.float32),
    ),
    mesh=_sc_mesh(),
    scratch_types=[
        pltpu.VMEM((U_CH, U), jnp.int32),
        pltpu.VMEM((U_CH, U), jnp.int32),
        pltpu.VMEM((U, KH), jnp.float32),
        pltpu.VMEM((U, KH), jnp.float32),
        pltpu.VMEM((U, KH), jnp.float32),
        pltpu.VMEM((U, KH), jnp.float32),
        pltpu.VMEM_SHARED((ACC_ROWS, KH), jnp.float32),
        pltpu.SemaphoreType.DMA,
        pltpu.SemaphoreType.DMA,
        pltpu.SemaphoreType.DMA,
        pltpu.SemaphoreType.DMA,
        pltpu.SemaphoreType.DMA,
        pltpu.SemaphoreType.DMA,
        pltpu.SemaphoreType.DMA,
        pltpu.SemaphoreType.DMA,
    ],
    compiler_params=_SC_PARAMS,
)


TC_B = 200
TC_GRID = (NC, HALF // TC_B)


def _deg_spec():
    return pl.BlockSpec((1, TC_B, DEG_W), lambda h, i: (h, i, 0))


def _ego_spec():
    return pl.BlockSpec((1, TC_B, K), lambda h, i: (h, i, 0))


def _col_spec():
    return pl.BlockSpec((NC, TC_B, KH), lambda h, i: (0, h * (HALF // TC_B) + i, 0))


def _flat_spec():
    return pl.BlockSpec((TC_B, KH), lambda h, i: (h * (HALF // TC_B) + i, 0))


def _prep_body(deg_ref, ego_ref, xh0_ref, dinv_ref):
    d = jnp.maximum(deg_ref[0, :, 0:1], 1.0)
    rs = lax.rsqrt(d)
    xh0_ref[0] = ego_ref[0, :, :KH] * rs
    xh0_ref[1] = ego_ref[0, :, KH:] * rs
    dinv_ref[...] = jnp.broadcast_to(1.0 / d, (TC_B, KH))


def _final_body(deg_ref, ego_ref, xh1_ref, xh2_ref, z3_ref, out_ref):
    d = jnp.maximum(deg_ref[0, :, 0:1], 1.0)
    rs = lax.rsqrt(d)
    for p in range(NC):
        zs = (xh1_ref[p] + xh2_ref[p]) * d + z3_ref[p]
        out_ref[0, :, p * KH:(p + 1) * KH] = (
            ego_ref[0, :, p * KH:(p + 1) * KH] + zs * rs) * 0.25


_prep_call = pl.pallas_call(
    _prep_body,
    grid=TC_GRID,
    in_specs=[_deg_spec(), _ego_spec()],
    out_specs=(_col_spec(), _flat_spec()),
    out_shape=(
        jax.ShapeDtypeStruct((NC, TBL_ROWS, KH), jnp.float32),
        jax.ShapeDtypeStruct((N_NODES, KH), jnp.float32),
    ),
)

_final_call = pl.pallas_call(
    _final_body,
    grid=TC_GRID,
    in_specs=[_deg_spec(), _ego_spec(), _col_spec(), _col_spec(), _col_spec()],
    out_specs=_ego_spec(),
    out_shape=jax.ShapeDtypeStruct((NC, HALF, K), jnp.float32),
)


def kernel(edge_index, Gu, Gi):
    e0 = edge_index[0].astype(jnp.int32)
    e1 = edge_index[1].astype(jnp.int32)
    npad = UNITS * U - e0.shape[0]
    ar = jnp.arange(npad, dtype=jnp.int32)
    pe0 = jnp.concatenate([e0, N_NODES + (ar % 64)])
    pe1 = jnp.concatenate([e1, N_NODES + 64 + (ar % 64)])
    ep = jnp.stack([pe0, pe1]).reshape(2, UNITS, U)
    ego = jnp.concatenate([Gu, Gi], axis=0)
    ego3d = ego.reshape(NC, HALF, K)

    deg = _deg_call(ep)
    xh0, dinv = _prep_call(deg, ego3d)
    xh1, xh2, z3 = _mega_call(ep, xh0, dinv)
    out = _final_call(deg, ego3d, xh1, xh2, z3)
    out = out.reshape(N_NODES, K)
    return out[:N_USERS], out[N_USERS:]


# speedup vs baseline: 1.2114x; 1.1244x over previous
import jax
import jax.numpy as jnp
from jax import lax
from jax.experimental import pallas as pl
from jax.experimental.pallas import tpu as pltpu
from jax.experimental.pallas import tpu_sc as plsc

N_USERS = 20000
N_ITEMS = 30000
N_NODES = N_USERS + N_ITEMS
K = 64
KH = 32

NC = 2
NS = 16

U = 128
UNITS = 6400
U_PER_TILE = UNITS // NS
U_CH = 50
N_CH = U_PER_TILE // U_CH
NB = 4
GRP = (2 * U_CH) // NB

ACC_ROWS = 50176
ZCH = 112
RO_CH = 125
NPT = N_NODES // NS


def _sc_mesh():
    return plsc.VectorSubcoreMesh(core_axis_name="c", subcore_axis_name="s")


_SC_PARAMS = pltpu.CompilerParams(use_tc_tiling_on_sc=False,
                                  needs_layout_passes=False)


def _rsqrt16(d):
    i = plsc.bitcast(d, jnp.int32)
    magic = jnp.full((16,), 0x5F3759DF, jnp.int32)
    one = jnp.full((16,), 1, jnp.int32)
    y = plsc.bitcast(magic - lax.shift_right_logical(i, one), jnp.float32)
    for _ in range(3):
        y = y * (1.5 - 0.5 * d * y * y)
    return y


def _mega_body(ep_hbm, ego_hbm, out_hbm, xh0_hbm, xh1_hbm, xh2_hbm,
               rs_hbm, dinv_hbm, zsum_hbm,
               e0_v, e1_v, r0_v, r1_v, r2_v, r3_v, acc,
               sem0, sem1, sem2, sem3):
    h = lax.axis_index("c")
    s = lax.axis_index("s")
    rows = (r0_v, r1_v, r2_v, r3_v)
    sems = (sem0, sem1, sem2, sem3)

    def fill_r3(val):
        v = jnp.full((16,), val, jnp.float32)

        def fr(r, _):
            for c in range(KH // 16):
                r3_v[r, pl.ds(c * 16, 16)] = v
            return 0

        lax.fori_loop(0, U, fr, 0)

    fill_r3(0.0)

    def zcp(i, _):
        pltpu.sync_copy(r3_v.at[pl.ds(0, ZCH)],
                        acc.at[pl.ds(s * (ACC_ROWS // NS) + i * ZCH, ZCH)])
        return 0

    lax.fori_loop(0, ACC_ROWS // (NS * ZCH), zcp, 0)
    plsc.subcore_barrier()

    fill_r3(1.0)

    def deg_chunk(oc, _):
        u0 = s * U_PER_TILE + oc * U_CH
        pltpu.sync_copy(ep_hbm.at[0, pl.ds(u0, U_CH)], e0_v)
        pltpu.sync_copy(ep_hbm.at[1, pl.ds(u0, U_CH)], e1_v)

        def du(k, _):
            pltpu.sync_copy(r3_v, acc.at[e1_v.at[k]], add=True)
            pltpu.sync_copy(r3_v, acc.at[e0_v.at[k]], add=True)
            return 0

        lax.fori_loop(0, U_CH, du, 0)
        return 0

    lax.fori_loop(0, N_CH, deg_chunk, 0)
    plsc.subcore_barrier()

    fill_r3(0.0)

    def prep_chunk(i, _):
        r0 = s * NPT + i * RO_CH
        pltpu.sync_copy(acc.at[pl.ds(r0, RO_CH)], r1_v.at[pl.ds(0, RO_CH)])

        def prow(r, _):
            d = jnp.maximum(r1_v[r, pl.ds(0, 16)], 1.0)
            y = _rsqrt16(d)
            yy = y * y
            for c in range(KH // 16):
                r0_v[r, pl.ds(c * 16, 16)] = y
                r1_v[r, pl.ds(c * 16, 16)] = yy
            return 0

        lax.fori_loop(0, RO_CH, prow, 0)
        pltpu.sync_copy(r0_v.at[pl.ds(0, RO_CH)],
                        rs_hbm.at[h, pl.ds(r0, RO_CH)])
        pltpu.sync_copy(r1_v.at[pl.ds(0, RO_CH)],
                        dinv_hbm.at[h, pl.ds(r0, RO_CH)])
        pltpu.sync_copy(ego_hbm.at[pl.ds(r0, RO_CH), pl.ds(h * KH, KH)],
                        r2_v.at[pl.ds(0, RO_CH)])

        def mrow(r, _):
            for c in range(KH // 16):
                sl = pl.ds(c * 16, 16)
                r2_v[r, sl] = r2_v[r, sl] * r0_v[r, sl]
            return 0

        lax.fori_loop(0, RO_CH, mrow, 0)
        pltpu.sync_copy(r2_v.at[pl.ds(0, RO_CH)],
                        xh0_hbm.at[h, pl.ds(r0, RO_CH)])
        pltpu.sync_copy(r3_v.at[pl.ds(0, RO_CH)], acc.at[pl.ds(r0, RO_CH)])
        return 0

    lax.fori_loop(0, N_NODES // (NS * RO_CH), prep_chunk, 0)
    plsc.subcore_barrier()

    def scatter_phase(table_hbm):
        def chunk(oc, _):
            u0 = s * U_PER_TILE + oc * U_CH
            pltpu.sync_copy(ep_hbm.at[0, pl.ds(u0, U_CH)], e0_v)
            pltpu.sync_copy(ep_hbm.at[1, pl.ds(u0, U_CH)], e1_v)

            def fire(g, j):
                k = 2 * g + (j // 2)
                gsrc = e0_v if j % 2 == 0 else e1_v
                pltpu.async_copy(table_hbm.at[gsrc.at[k]], rows[j], sems[j])

            def scat(g, j):
                k = 2 * g + (j // 2)
                gdst = e1_v if j % 2 == 0 else e0_v
                pltpu.sync_copy(rows[j], acc.at[gdst.at[k]], add=True)

            for j in range(NB):
                fire(0, j)

            def grp(g, _):
                for j in range(NB):
                    pltpu.make_async_copy(
                        table_hbm.at[pl.ds(0, U)], rows[j], sems[j]
                    ).wait()
                    scat(g, j)

                    @pl.when(g < GRP - 1)
                    def _():
                        fire(g + 1, j)

                return 0

            lax.fori_loop(0, GRP, grp, 0)
            return 0

        lax.fori_loop(0, N_CH, chunk, 0)
        plsc.subcore_barrier()

    def readout_phase(out_tbl_hbm, first):
        fill_r3(0.0)

        def ro(i, _):
            r0 = s * NPT + i * RO_CH
            sl = pl.ds(0, RO_CH)
            pltpu.sync_copy(acc.at[pl.ds(r0, RO_CH)], r0_v.at[sl])
            if first:
                pltpu.sync_copy(r0_v.at[sl], zsum_hbm.at[h, pl.ds(r0, RO_CH)])
                pltpu.sync_copy(dinv_hbm.at[h, pl.ds(r0, RO_CH)], r1_v.at[sl])

                def mr(r, _):
                    for c in range(KH // 16):
                        cc = pl.ds(c * 16, 16)
                        r0_v[r, cc] = r0_v[r, cc] * r1_v[r, cc]
                    return 0

                lax.fori_loop(0, RO_CH, mr, 0)
            else:
                pltpu.sync_copy(zsum_hbm.at[h, pl.ds(r0, RO_CH)], r1_v.at[sl])
                pltpu.sync_copy(dinv_hbm.at[h, pl.ds(r0, RO_CH)], r2_v.at[sl])

                def mr(r, _):
                    for c in range(KH // 16):
                        cc = pl.ds(c * 16, 16)
                        z = r0_v[r, cc]
                        r1_v[r, cc] = r1_v[r, cc] + z
                        r0_v[r, cc] = z * r2_v[r, cc]
                    return 0

                lax.fori_loop(0, RO_CH, mr, 0)
                pltpu.sync_copy(r1_v.at[sl], zsum_hbm.at[h, pl.ds(r0, RO_CH)])
            pltpu.sync_copy(r0_v.at[sl], out_tbl_hbm.at[h, pl.ds(r0, RO_CH)])
            pltpu.sync_copy(r3_v.at[sl], acc.at[pl.ds(r0, RO_CH)])
            return 0

        lax.fori_loop(0, N_NODES // (NS * RO_CH), ro, 0)
        plsc.subcore_barrier()

    scatter_phase(xh0_hbm.at[h])
    readout_phase(xh1_hbm, first=True)
    scatter_phase(xh1_hbm.at[h])
    readout_phase(xh2_hbm, first=False)
    scatter_phase(xh2_hbm.at[h])

    def fin(i, _):
        r0 = s * NPT + i * RO_CH
        sl = pl.ds(0, RO_CH)
        pltpu.sync_copy(acc.at[pl.ds(r0, RO_CH)], r0_v.at[sl])
        pltpu.sync_copy(zsum_hbm.at[h, pl.ds(r0, RO_CH)], r1_v.at[sl])
        pltpu.sync_copy(rs_hbm.at[h, pl.ds(r0, RO_CH)], r2_v.at[sl])
        pltpu.sync_copy(ego_hbm.at[pl.ds(r0, RO_CH), pl.ds(h * KH, KH)],
                        r3_v.at[sl])

        def fr(r, _):
            for c in range(KH // 16):
                cc = pl.ds(c * 16, 16)
                r1_v[r, cc] = (r3_v[r, cc]
                               + (r1_v[r, cc] + r0_v[r, cc]) * r2_v[r, cc]
                               ) * 0.25
            return 0

        lax.fori_loop(0, RO_CH, fr, 0)
        pltpu.sync_copy(r1_v.at[sl],
                        out_hbm.at[pl.ds(r0, RO_CH), pl.ds(h * KH, KH)])
        return 0

    lax.fori_loop(0, N_NODES // (NS * RO_CH), fin, 0)


_mega_call = pl.kernel(
    _mega_body,
    out_type=(
        jax.ShapeDtypeStruct((N_NODES, K), jnp.float32),
        jax.ShapeDtypeStruct((NC, ACC_ROWS, KH), jnp.float32),
        jax.ShapeDtypeStruct((NC, ACC_ROWS, KH), jnp.float32),
        jax.ShapeDtypeStruct((NC, ACC_ROWS, KH), jnp.float32),
        jax.ShapeDtypeStruct((NC, N_NODES, KH), jnp.float32),
        jax.ShapeDtypeStruct((NC, N_NODES, KH), jnp.float32),
        jax.ShapeDtypeStruct((NC, N_NODES, KH), jnp.float32),
    ),
    mesh=_sc_mesh(),
    scratch_types=[
        pltpu.VMEM((U_CH, U), jnp.int32),
        pltpu.VMEM((U_CH, U), jnp.int32),
        pltpu.VMEM((U, KH), jnp.float32),
        pltpu.VMEM((U, KH), jnp.float32),
        pltpu.VMEM((U, KH), jnp.float32),
        pltpu.VMEM((U, KH), jnp.float32),
        pltpu.VMEM_SHARED((ACC_ROWS, KH), jnp.float32),
        pltpu.SemaphoreType.DMA,
        pltpu.SemaphoreType.DMA,
        pltpu.SemaphoreType.DMA,
        pltpu.SemaphoreType.DMA,
    ],
    compiler_params=_SC_PARAMS,
)


def kernel(edge_index, Gu, Gi):
    e0 = edge_index[0].astype(jnp.int32)
    e1 = edge_index[1].astype(jnp.int32)
    npad = UNITS * U - e0.shape[0]
    ar = jnp.arange(npad, dtype=jnp.int32)
    pe0 = jnp.concatenate([e0, N_NODES + (ar % 64)])
    pe1 = jnp.concatenate([e1, N_NODES + 64 + (ar % 64)])
    ep = jnp.stack([pe0, pe1]).reshape(2, UNITS, U)
    ego = jnp.concatenate([Gu, Gi], axis=0)

    out = _mega_call(ep, ego)[0]
    return out[:N_USERS], out[N_USERS:]


# speedup vs baseline: 1.2406x; 1.0242x over previous
import jax
import jax.numpy as jnp
from jax import lax
from jax.experimental import pallas as pl
from jax.experimental.pallas import tpu as pltpu
from jax.experimental.pallas import tpu_sc as plsc

N_USERS = 20000
N_ITEMS = 30000
N_NODES = N_USERS + N_ITEMS
K = 64
KH = 32

NC = 2
NS = 16

U = 128
UNITS = 6400
U_PER_TILE = UNITS // NS
U_CH = 50
N_CH = U_PER_TILE // U_CH
NB = 4
GRP = (2 * U_CH) // NB

ACC_ROWS = 50176
ZCH = 112
RO_CH = 125
NPT = N_NODES // NS


def _sc_mesh():
    return plsc.VectorSubcoreMesh(core_axis_name="c", subcore_axis_name="s")


_SC_PARAMS = pltpu.CompilerParams(use_tc_tiling_on_sc=False,
                                  needs_layout_passes=False)


def _rsqrt16(d):
    i = plsc.bitcast(d, jnp.int32)
    magic = jnp.full((16,), 0x5F3759DF, jnp.int32)
    one = jnp.full((16,), 1, jnp.int32)
    y = plsc.bitcast(magic - lax.shift_right_logical(i, one), jnp.float32)
    for _ in range(3):
        y = y * (1.5 - 0.5 * d * y * y)
    return y


def _mega_body(ep_hbm, ego_hbm, out_hbm, xh0_hbm, xh1_hbm, xh2_hbm,
               rs_hbm, dinv_hbm, zsum_hbm,
               e0_v, e1_v, r0_v, r1_v, r2_v, r3_v, acc,
               sem0, sem1, sem2, sem3):
    h = lax.axis_index("c")
    s = lax.axis_index("s")
    rows = (r0_v, r1_v, r2_v, r3_v)
    sems = (sem0, sem1, sem2, sem3)

    def fill_r3(val):
        v = jnp.full((16,), val, jnp.float32)

        def fr(r, _):
            for c in range(KH // 16):
                r3_v[r, pl.ds(c * 16, 16)] = v
            return 0

        lax.fori_loop(0, U, fr, 0)

    fill_r3(0.0)

    def zcp(i, _):
        pltpu.sync_copy(r3_v.at[pl.ds(0, ZCH)],
                        acc.at[pl.ds(s * (ACC_ROWS // NS) + i * ZCH, ZCH)])
        return 0

    lax.fori_loop(0, ACC_ROWS // (NS * ZCH), zcp, 0)
    plsc.subcore_barrier()

    fill_r3(1.0)

    def deg_chunk(oc, _):
        u0 = s * U_PER_TILE + oc * U_CH
        pltpu.sync_copy(ep_hbm.at[0, pl.ds(u0, U_CH)], e0_v)
        pltpu.sync_copy(ep_hbm.at[1, pl.ds(u0, U_CH)], e1_v)

        def dfire(g, j):
            k = 2 * g + (j // 2)
            gdst = e1_v if j % 2 == 0 else e0_v
            pltpu.async_copy(r3_v, acc.at[gdst.at[k]], sems[j], add=True)

        def dwait(j):
            pltpu.make_async_copy(r3_v, acc.at[pl.ds(0, U)], sems[j]).wait()

        for j in range(NB):
            dfire(0, j)

        def dgrp(g, _):
            for j in range(NB):
                dwait(j)

                @pl.when(g < GRP - 1)
                def _():
                    dfire(g + 1, j)

            return 0

        lax.fori_loop(0, GRP, dgrp, 0)
        return 0

    lax.fori_loop(0, N_CH, deg_chunk, 0)
    plsc.subcore_barrier()

    fill_r3(0.0)

    def prep_chunk(i, _):
        r0 = s * NPT + i * RO_CH
        pltpu.sync_copy(acc.at[pl.ds(r0, RO_CH)], r1_v.at[pl.ds(0, RO_CH)])

        def prow(r, _):
            d = jnp.maximum(r1_v[r, pl.ds(0, 16)], 1.0)
            y = _rsqrt16(d)
            yy = y * y
            for c in range(KH // 16):
                r0_v[r, pl.ds(c * 16, 16)] = y
                r1_v[r, pl.ds(c * 16, 16)] = yy
            return 0

        lax.fori_loop(0, RO_CH, prow, 0)
        pltpu.sync_copy(r0_v.at[pl.ds(0, RO_CH)],
                        rs_hbm.at[h, pl.ds(r0, RO_CH)])
        pltpu.sync_copy(r1_v.at[pl.ds(0, RO_CH)],
                        dinv_hbm.at[h, pl.ds(r0, RO_CH)])
        pltpu.sync_copy(ego_hbm.at[pl.ds(r0, RO_CH), pl.ds(h * KH, KH)],
                        r2_v.at[pl.ds(0, RO_CH)])

        def mrow(r, _):
            for c in range(KH // 16):
                sl = pl.ds(c * 16, 16)
                r2_v[r, sl] = r2_v[r, sl] * r0_v[r, sl]
            return 0

        lax.fori_loop(0, RO_CH, mrow, 0)
        pltpu.sync_copy(r2_v.at[pl.ds(0, RO_CH)],
                        xh0_hbm.at[h, pl.ds(r0, RO_CH)])
        pltpu.sync_copy(r3_v.at[pl.ds(0, RO_CH)], acc.at[pl.ds(r0, RO_CH)])
        return 0

    lax.fori_loop(0, N_NODES // (NS * RO_CH), prep_chunk, 0)
    plsc.subcore_barrier()

    def scatter_phase(table_hbm):
        def chunk(oc, _):
            u0 = s * U_PER_TILE + oc * U_CH
            pltpu.sync_copy(ep_hbm.at[0, pl.ds(u0, U_CH)], e0_v)
            pltpu.sync_copy(ep_hbm.at[1, pl.ds(u0, U_CH)], e1_v)

            def fire(g, j):
                k = 2 * g + (j // 2)
                gsrc = e0_v if j % 2 == 0 else e1_v
                pltpu.async_copy(table_hbm.at[gsrc.at[k]], rows[j], sems[j])

            def scat(g, j):
                k = 2 * g + (j // 2)
                gdst = e1_v if j % 2 == 0 else e0_v
                pltpu.sync_copy(rows[j], acc.at[gdst.at[k]], add=True)

            for j in range(NB):
                fire(0, j)

            def grp(g, _):
                for j in range(NB):
                    pltpu.make_async_copy(
                        table_hbm.at[pl.ds(0, U)], rows[j], sems[j]
                    ).wait()
                    scat(g, j)

                    @pl.when(g < GRP - 1)
                    def _():
                        fire(g + 1, j)

                return 0

            lax.fori_loop(0, GRP, grp, 0)
            return 0

        lax.fori_loop(0, N_CH, chunk, 0)
        plsc.subcore_barrier()

    def readout_phase(out_tbl_hbm, first):
        fill_r3(0.0)

        def ro(i, _):
            r0 = s * NPT + i * RO_CH
            sl = pl.ds(0, RO_CH)
            pltpu.sync_copy(acc.at[pl.ds(r0, RO_CH)], r0_v.at[sl])
            if first:
                pltpu.sync_copy(r0_v.at[sl], zsum_hbm.at[h, pl.ds(r0, RO_CH)])
                pltpu.sync_copy(dinv_hbm.at[h, pl.ds(r0, RO_CH)], r1_v.at[sl])

                def mr(r, _):
                    for c in range(KH // 16):
                        cc = pl.ds(c * 16, 16)
                        r0_v[r, cc] = r0_v[r, cc] * r1_v[r, cc]
                    return 0

                lax.fori_loop(0, RO_CH, mr, 0)
            else:
                pltpu.sync_copy(zsum_hbm.at[h, pl.ds(r0, RO_CH)], r1_v.at[sl])
                pltpu.sync_copy(dinv_hbm.at[h, pl.ds(r0, RO_CH)], r2_v.at[sl])

                def mr(r, _):
                    for c in range(KH // 16):
                        cc = pl.ds(c * 16, 16)
                        z = r0_v[r, cc]
                        r1_v[r, cc] = r1_v[r, cc] + z
                        r0_v[r, cc] = z * r2_v[r, cc]
                    return 0

                lax.fori_loop(0, RO_CH, mr, 0)
                pltpu.sync_copy(r1_v.at[sl], zsum_hbm.at[h, pl.ds(r0, RO_CH)])
            pltpu.sync_copy(r0_v.at[sl], out_tbl_hbm.at[h, pl.ds(r0, RO_CH)])
            pltpu.sync_copy(r3_v.at[sl], acc.at[pl.ds(r0, RO_CH)])
            return 0

        lax.fori_loop(0, N_NODES // (NS * RO_CH), ro, 0)
        plsc.subcore_barrier()

    scatter_phase(xh0_hbm.at[h])
    readout_phase(xh1_hbm, first=True)
    scatter_phase(xh1_hbm.at[h])
    readout_phase(xh2_hbm, first=False)
    scatter_phase(xh2_hbm.at[h])

    def fin(i, _):
        r0 = s * NPT + i * RO_CH
        sl = pl.ds(0, RO_CH)
        pltpu.sync_copy(acc.at[pl.ds(r0, RO_CH)], r0_v.at[sl])
        pltpu.sync_copy(zsum_hbm.at[h, pl.ds(r0, RO_CH)], r1_v.at[sl])
        pltpu.sync_copy(rs_hbm.at[h, pl.ds(r0, RO_CH)], r2_v.at[sl])
        pltpu.sync_copy(ego_hbm.at[pl.ds(r0, RO_CH), pl.ds(h * KH, KH)],
                        r3_v.at[sl])

        def fr(r, _):
            for c in range(KH // 16):
                cc = pl.ds(c * 16, 16)
                r1_v[r, cc] = (r3_v[r, cc]
                               + (r1_v[r, cc] + r0_v[r, cc]) * r2_v[r, cc]
                               ) * 0.25
            return 0

        lax.fori_loop(0, RO_CH, fr, 0)
        pltpu.sync_copy(r1_v.at[sl],
                        out_hbm.at[pl.ds(r0, RO_CH), pl.ds(h * KH, KH)])
        return 0

    lax.fori_loop(0, N_NODES // (NS * RO_CH), fin, 0)


_mega_call = pl.kernel(
    _mega_body,
    out_type=(
        jax.ShapeDtypeStruct((N_NODES, K), jnp.float32),
        jax.ShapeDtypeStruct((NC, ACC_ROWS, KH), jnp.float32),
        jax.ShapeDtypeStruct((NC, ACC_ROWS, KH), jnp.float32),
        jax.ShapeDtypeStruct((NC, ACC_ROWS, KH), jnp.float32),
        jax.ShapeDtypeStruct((NC, N_NODES, KH), jnp.float32),
        jax.ShapeDtypeStruct((NC, N_NODES, KH), jnp.float32),
        jax.ShapeDtypeStruct((NC, N_NODES, KH), jnp.float32),
    ),
    mesh=_sc_mesh(),
    scratch_types=[
        pltpu.VMEM((U_CH, U), jnp.int32),
        pltpu.VMEM((U_CH, U), jnp.int32),
        pltpu.VMEM((U, KH), jnp.float32),
        pltpu.VMEM((U, KH), jnp.float32),
        pltpu.VMEM((U, KH), jnp.float32),
        pltpu.VMEM((U, KH), jnp.float32),
        pltpu.VMEM_SHARED((ACC_ROWS, KH), jnp.float32),
        pltpu.SemaphoreType.DMA,
        pltpu.SemaphoreType.DMA,
        pltpu.SemaphoreType.DMA,
        pltpu.SemaphoreType.DMA,
    ],
    compiler_params=_SC_PARAMS,
)


def kernel(edge_index, Gu, Gi):
    e0 = edge_index[0].astype(jnp.int32)
    e1 = edge_index[1].astype(jnp.int32)
    npad = UNITS * U - e0.shape[0]
    ar = jnp.arange(npad, dtype=jnp.int32)
    pe0 = jnp.concatenate([e0, N_NODES + (ar % 64)])
    pe1 = jnp.concatenate([e1, N_NODES + 64 + (ar % 64)])
    ep = jnp.stack([pe0, pe1]).reshape(2, UNITS, U)
    ego = jnp.concatenate([Gu, Gi], axis=0)

    out = _mega_call(ep, ego)[0]
    return out[:N_USERS], out[N_USERS:]
